# Initial kernel scaffold; baseline (speedup 1.0000x reference)
import jax
import jax.numpy as jnp
from jax.experimental import pallas as pl


def kernel(edge_index, edge_vals, user_emb, item_emb):
    raise NotImplementedError("write your pallas kernel here")


# speedup vs baseline: 1.5095x; 1.5095x over previous
import functools

import jax
import jax.numpy as jnp
from jax import lax
from jax.experimental import pallas as pl
from jax.experimental.pallas import tpu as pltpu
from jax.experimental.pallas import tpu_sc as plsc

_NUM_USERS = 5000
_NUM_ITEMS = 5000
_N_EDGES = 160000
_D = 256
_N_LAYERS = 3

_HALF = 5000
_PAD = 8
_HALF_P = _HALF + _PAD
_N_P = 2 * _HALF_P

_NC = 2
_NS = 16
_LANES = 16
_EPT = _N_EDGES // _NS
_G = 80
_NB = _EPT // _G
_STRIPE = 320
_LAST_ROW0 = _HALF_P - _STRIPE


def _layer_body(src_hbm, dst_hbm, vrep_hbm, emb_hbm, out_hbm,
                src_raw, dst_raw, src_b, dst_b, vrep_b,
                rows, acc, sem):
    c = lax.axis_index("c")
    s = lax.axis_index("s")

    zvec = jnp.zeros((_LANES,), jnp.float32)

    def _zero_row(r, _):
        for j in range(_D // _LANES):
            rows[r, pl.ds(j * _LANES, _LANES)] = zvec
        return 0

    lax.fori_loop(0, _G, _zero_row, 0)
    row0 = lax.min(_STRIPE * s, jnp.int32(_LAST_ROW0))
    for off in range(0, _STRIPE, _G):
        pltpu.sync_copy(rows, acc.at[pl.ds(row0 + off, _G)])
    plsc.subcore_barrier()

    lane_pad = _HALF + lax.rem(lax.iota(jnp.int32, _LANES), jnp.int32(_PAD))
    dst_off = c * _HALF
    e0 = s * _EPT

    def _batch(b, _):
        base = e0 + b * _G
        pltpu.sync_copy(src_hbm.at[pl.ds(base, _G)], src_raw)
        pltpu.sync_copy(dst_hbm.at[pl.ds(base, _G)], dst_raw)
        pltpu.sync_copy(vrep_hbm.at[pl.ds(base, _G)], vrep_b)
        for k in range(_G // _LANES):
            sv = src_raw[pl.ds(k * _LANES, _LANES)]
            sv = sv + jnp.where(sv >= _HALF, jnp.int32(_PAD), jnp.int32(0))
            src_b[pl.ds(k * _LANES, _LANES)] = sv
            dv = dst_raw[pl.ds(k * _LANES, _LANES)] - dst_off
            ok = (dv >= 0) & (dv < _HALF)
            dst_b[pl.ds(k * _LANES, _LANES)] = jnp.where(ok, dv, lane_pad)
        pltpu.async_copy(emb_hbm.at[src_b], rows, sem).wait()

        def _scale(e, _):
            v = vrep_b[e]
            for j in range(_D // _LANES):
                rows[e, pl.ds(j * _LANES, _LANES)] = (
                    rows[e, pl.ds(j * _LANES, _LANES)] * v)
            return 0

        lax.fori_loop(0, _G, _scale, 0)
        pltpu.sync_copy(rows, acc.at[dst_b], add=True)
        return 0

    lax.fori_loop(0, _NB, _batch, 0)
    plsc.subcore_barrier()

    pltpu.sync_copy(acc.at[pl.ds(row0, _STRIPE)],
                    out_hbm.at[pl.ds(c * _HALF_P + row0, _STRIPE)])


_layer = functools.partial(
    pl.kernel,
    mesh=plsc.VectorSubcoreMesh(core_axis_name="c", subcore_axis_name="s"),
    compiler_params=pltpu.CompilerParams(use_tc_tiling_on_sc=False),
    out_type=jax.ShapeDtypeStruct((_N_P, _D), jnp.float32),
    scratch_types=[
        pltpu.VMEM((_G,), jnp.int32),
        pltpu.VMEM((_G,), jnp.int32),
        pltpu.VMEM((_G,), jnp.int32),
        pltpu.VMEM((_G,), jnp.int32),
        pltpu.VMEM((_G, _LANES), jnp.float32),
        pltpu.VMEM((_G, _D), jnp.float32),
        pltpu.VMEM_SHARED((_HALF_P, _D), jnp.float32),
        pltpu.SemaphoreType.DMA,
    ],
)(_layer_body)


def _mean_body(a_ref, b_ref, c_ref, d_ref, o_ref):
    o_ref[...] = (a_ref[...] + b_ref[...] + c_ref[...] + d_ref[...]) * 0.25


_mean = pl.pallas_call(
    _mean_body,
    grid=(_N_P // 16,),
    in_specs=[pl.BlockSpec((16, _D), lambda i: (i, 0))] * 4,
    out_specs=pl.BlockSpec((16, _D), lambda i: (i, 0)),
    out_shape=jax.ShapeDtypeStruct((_N_P, _D), jnp.float32),
)


def kernel(edge_index, edge_vals, user_emb, item_emb):
    src = edge_index[0]
    dst = edge_index[1]
    vrep = jnp.broadcast_to(edge_vals[:, None], (_N_EDGES, _LANES))
    zpad = jnp.zeros((_PAD, _D), jnp.float32)
    emb = jnp.concatenate([user_emb, zpad, item_emb, zpad], axis=0)
    embs = [emb]
    for _ in range(_N_LAYERS):
        emb = _layer(src, dst, vrep, emb)
        embs.append(emb)
    light = _mean(*embs)
    users = light[:_NUM_USERS]
    items = light[_HALF_P:_HALF_P + _NUM_ITEMS]
    return (users, items)


# speedup vs baseline: 2.3460x; 1.5542x over previous
import functools

import jax
import jax.numpy as jnp
from jax import lax
from jax.experimental import pallas as pl
from jax.experimental.pallas import tpu as pltpu
from jax.experimental.pallas import tpu_sc as plsc

_NUM_USERS = 5000
_NUM_ITEMS = 5000
_N_EDGES = 160000
_D = 256
_N_LAYERS = 3

_HALF = 5000
_PAD = 8
_HALF_P = _HALF + _PAD
_N_P = 2 * _HALF_P

_NC = 2
_NS = 16
_LANES = 16
_EPT = _N_EDGES // _NS
_G = 80
_NB = _EPT // _G
_STRIPE = 320
_LAST_ROW0 = _HALF_P - _STRIPE


def _layer_body(src_hbm, dst_hbm, vrep_hbm, emb_hbm, out_hbm,
                src_raw0, dst_raw0, src_b0, dst_b0, vrep_b0, rows0,
                src_raw1, dst_raw1, src_b1, dst_b1, vrep_b1, rows1,
                acc, gsem0, gsem1, ssem):
    c = lax.axis_index("c")
    s = lax.axis_index("s")

    bufs0 = (src_raw0, dst_raw0, src_b0, dst_b0, vrep_b0, rows0, gsem0)
    bufs1 = (src_raw1, dst_raw1, src_b1, dst_b1, vrep_b1, rows1, gsem1)

    zvec = jnp.zeros((_LANES,), jnp.float32)

    def _zero_row(r, _):
        for j in range(_D // _LANES):
            rows0[r, pl.ds(j * _LANES, _LANES)] = zvec
        return 0

    lax.fori_loop(0, _G, _zero_row, 0)
    row0 = lax.min(_STRIPE * s, jnp.int32(_LAST_ROW0))
    for off in range(0, _STRIPE, _G):
        pltpu.sync_copy(rows0, acc.at[pl.ds(row0 + off, _G)])
    plsc.subcore_barrier()

    lane_pad = _HALF + lax.rem(lax.iota(jnp.int32, _LANES), jnp.int32(_PAD))
    dst_off = c * _HALF
    e0 = s * _EPT

    def _stage_and_gather(b, bufs):
        src_raw, dst_raw, src_b, dst_b, vrep_b, rows, gsem = bufs
        base = e0 + b * _G
        h1 = pltpu.async_copy(src_hbm.at[pl.ds(base, _G)], src_raw, ssem)
        h2 = pltpu.async_copy(dst_hbm.at[pl.ds(base, _G)], dst_raw, ssem)
        h3 = pltpu.async_copy(vrep_hbm.at[pl.ds(base, _G)], vrep_b, ssem)
        h1.wait()
        h2.wait()
        h3.wait()
        for k in range(_G // _LANES):
            sv = src_raw[pl.ds(k * _LANES, _LANES)]
            sv = sv + jnp.where(sv >= _HALF, jnp.int32(_PAD), jnp.int32(0))
            src_b[pl.ds(k * _LANES, _LANES)] = sv
            dv = dst_raw[pl.ds(k * _LANES, _LANES)] - dst_off
            ok = (dv >= 0) & (dv < _HALF)
            dst_b[pl.ds(k * _LANES, _LANES)] = jnp.where(ok, dv, lane_pad)
        pltpu.async_copy(emb_hbm.at[src_b], rows, gsem)

    def _process(bufs):
        src_raw, dst_raw, src_b, dst_b, vrep_b, rows, gsem = bufs
        pltpu.make_async_copy(emb_hbm.at[src_b], rows, gsem).wait()

        def _scale(e, _):
            v = vrep_b[e]
            for j in range(_D // _LANES):
                rows[e, pl.ds(j * _LANES, _LANES)] = (
                    rows[e, pl.ds(j * _LANES, _LANES)] * v)
            return 0

        lax.fori_loop(0, _G, _scale, 0)
        pltpu.sync_copy(rows, acc.at[dst_b], add=True)

    _stage_and_gather(jnp.int32(0), bufs0)

    def _pair(t, _):
        b = 2 * t
        _stage_and_gather(b + 1, bufs1)
        _process(bufs0)
        _stage_and_gather(b + 2, bufs0)
        _process(bufs1)
        return 0

    lax.fori_loop(0, (_NB - 1) // 2, _pair, 0)
    _process(bufs0)
    plsc.subcore_barrier()

    pltpu.sync_copy(acc.at[pl.ds(row0, _STRIPE)],
                    out_hbm.at[pl.ds(c * _HALF_P + row0, _STRIPE)])


_layer = functools.partial(
    pl.kernel,
    mesh=plsc.VectorSubcoreMesh(core_axis_name="c", subcore_axis_name="s"),
    compiler_params=pltpu.CompilerParams(use_tc_tiling_on_sc=False),
    out_type=jax.ShapeDtypeStruct((_N_P, _D), jnp.float32),
    scratch_types=[
        pltpu.VMEM((_G,), jnp.int32),
        pltpu.VMEM((_G,), jnp.int32),
        pltpu.VMEM((_G,), jnp.int32),
        pltpu.VMEM((_G,), jnp.int32),
        pltpu.VMEM((_G, _LANES), jnp.float32),
        pltpu.VMEM((_G, _D), jnp.float32),
        pltpu.VMEM((_G,), jnp.int32),
        pltpu.VMEM((_G,), jnp.int32),
        pltpu.VMEM((_G,), jnp.int32),
        pltpu.VMEM((_G,), jnp.int32),
        pltpu.VMEM((_G, _LANES), jnp.float32),
        pltpu.VMEM((_G, _D), jnp.float32),
        pltpu.VMEM_SHARED((_HALF_P, _D), jnp.float32),
        pltpu.SemaphoreType.DMA,
        pltpu.SemaphoreType.DMA,
        pltpu.SemaphoreType.DMA,
    ],
)(_layer_body)


def _mean_body(a_ref, b_ref, c_ref, d_ref, o_ref):
    o_ref[...] = (a_ref[...] + b_ref[...] + c_ref[...] + d_ref[...]) * 0.25


_mean = pl.pallas_call(
    _mean_body,
    grid=(_N_P // 16,),
    in_specs=[pl.BlockSpec((16, _D), lambda i: (i, 0))] * 4,
    out_specs=pl.BlockSpec((16, _D), lambda i: (i, 0)),
    out_shape=jax.ShapeDtypeStruct((_N_P, _D), jnp.float32),
)


def kernel(edge_index, edge_vals, user_emb, item_emb):
    src = edge_index[0]
    dst = edge_index[1]
    vrep = jnp.broadcast_to(edge_vals[:, None], (_N_EDGES, _LANES))
    zpad = jnp.zeros((_PAD, _D), jnp.float32)
    emb = jnp.concatenate([user_emb, zpad, item_emb, zpad], axis=0)
    embs = [emb]
    for _ in range(_N_LAYERS):
        emb = _layer(src, dst, vrep, emb)
        embs.append(emb)
    light = _mean(*embs)
    users = light[:_NUM_USERS]
    items = light[_HALF_P:_HALF_P + _NUM_ITEMS]
    return (users, items)
